# baseline (speedup 1.0000x reference)
import functools

import jax
import jax.numpy as jnp
from jax import lax
from jax.experimental import pallas as pl
from jax.experimental.pallas import tpu as pltpu
from jax.experimental.pallas import tpu_sc as plsc

N = 10000
E = 320000
D = 128
G = 8

NC = 2
NS = 16
NW = NC * NS
EDGES_PER_TILE = E // NW
CHUNK = 112
NCHUNK = -(-EDGES_PER_TILE // CHUNK)
EDGES_PAD = NCHUNK * CHUNK - EDGES_PER_TILE
TRASH = N
NACC = N + 8
CHUNK_C = 80
NCHUNK_C = EDGES_PER_TILE // CHUNK_C
ROWS_PER_TILE = 624
ZERO_REM = NACC - NS * ROWS_PER_TILE
OUT_REM = N - NS * ROWS_PER_TILE

_BLK = 1000
_GRID = N // _BLK


def _agg_body(h_hbm, idx_hbm, zeros_hbm, out_hbm,
              ibuf, rows_a, accum, gsem_a):
    cid = lax.axis_index("c")
    sid = lax.axis_index("s")
    wid = cid * NS + sid

    pltpu.sync_copy(zeros_hbm.at[pl.ds(sid * ROWS_PER_TILE, ROWS_PER_TILE)],
                    accum.at[pl.ds(sid * ROWS_PER_TILE, ROWS_PER_TILE)])

    @pl.when(sid == NS - 1)
    def _():
        pltpu.sync_copy(zeros_hbm.at[pl.ds(NS * ROWS_PER_TILE, ZERO_REM)],
                        accum.at[pl.ds(NS * ROWS_PER_TILE, ZERO_REM)])

    pltpu.sync_copy(idx_hbm.at[wid], ibuf)
    plsc.subcore_barrier()

    def body(j, carry):
        pltpu.async_copy(h_hbm.at[ibuf.at[2 * j]], rows_a, gsem_a).wait()
        pltpu.sync_copy(rows_a, accum.at[ibuf.at[2 * j + 1]], add=True)
        return carry

    lax.fori_loop(0, NCHUNK, body, 0)
    plsc.subcore_barrier()
    pltpu.sync_copy(accum.at[pl.ds(sid * ROWS_PER_TILE, ROWS_PER_TILE)],
                    out_hbm.at[cid, pl.ds(sid * ROWS_PER_TILE, ROWS_PER_TILE)])

    @pl.when(sid == NS - 1)
    def _():
        pltpu.sync_copy(accum.at[pl.ds(NS * ROWS_PER_TILE, OUT_REM)],
                        out_hbm.at[cid, pl.ds(NS * ROWS_PER_TILE, OUT_REM)])


def _cnt_body(dst_hbm, zeros_hbm, out_hbm, idx_d, ones, cnt, sem):
    cid = lax.axis_index("c")
    sid = lax.axis_index("s")
    wid = cid * NS + sid

    @pl.when(sid == 0)
    def _():
        pltpu.sync_copy(zeros_hbm, cnt)
    for i in range(CHUNK_C // 16):
        ones[pl.ds(i * 16, 16)] = jnp.full((16,), 1.0, jnp.float32)
    pltpu.sync_copy(dst_hbm.at[wid], idx_d)
    plsc.subcore_barrier()

    def body(j, carry):
        pltpu.sync_copy(ones, cnt.at[idx_d.at[j]], add=True)
        return carry

    lax.fori_loop(0, NCHUNK_C, body, 0)
    plsc.subcore_barrier()

    @pl.when(sid == 0)
    def _():
        pltpu.sync_copy(cnt, out_hbm.at[cid, 0])


@functools.cache
def _sc_kernels():
    mesh = plsc.VectorSubcoreMesh(core_axis_name="c", subcore_axis_name="s",
                                  num_cores=NC, num_subcores=NS)
    agg = pl.kernel(
        _agg_body,
        out_type=jax.ShapeDtypeStruct((NC, N, D), jnp.float32),
        mesh=mesh,
        scratch_types=(
            [pltpu.VMEM((2 * NCHUNK, CHUNK), jnp.int32)]
            + [pltpu.VMEM((CHUNK, D), jnp.float32)]
            + [pltpu.VMEM_SHARED((NACC, D), jnp.float32)]
            + [pltpu.SemaphoreType.DMA]
        ),
    )
    cnt = pl.kernel(
        _cnt_body,
        out_type=jax.ShapeDtypeStruct((NC, 1, N), jnp.float32),
        mesh=mesh,
        scratch_types=[
            pltpu.VMEM((NCHUNK_C, CHUNK_C), jnp.int32),
            pltpu.VMEM((CHUNK_C,), jnp.float32),
            pltpu.VMEM_SHARED((N,), jnp.float32),
            pltpu.SemaphoreType.DMA,
        ],
    )
    return agg, cnt


def _ln_relu(y, g, be):
    m = jnp.mean(y, axis=-1, keepdims=True)
    v = jnp.mean((y - m) ** 2, axis=-1, keepdims=True)
    return jnp.maximum((y - m) * lax.rsqrt(v + 1e-5) * g + be, 0.0)


def _dense0_kernel(x_ref, w_ref, b_ref, g_ref, be_ref, o_ref):
    y = jnp.dot(x_ref[...], w_ref[...], preferred_element_type=jnp.float32)
    o_ref[...] = _ln_relu(y + b_ref[...], g_ref[...], be_ref[...])


def _sage_kernel(h_ref, p0_ref, p1_ref, c0_ref, c1_ref,
                 wl_ref, wr_ref, b_ref, g_ref, be_ref, o_ref):
    cnt = jnp.clip(c0_ref[...] + c1_ref[...], 1.0, None)
    nmean = (p0_ref[...] + p1_ref[...]) / cnt
    y = (jnp.dot(h_ref[...], wl_ref[...], preferred_element_type=jnp.float32)
         + jnp.dot(nmean, wr_ref[...], preferred_element_type=jnp.float32))
    o_ref[...] = _ln_relu(y + b_ref[...], g_ref[...], be_ref[...])


def _sage_pool_kernel(h_ref, p0_ref, p1_ref, c0_ref, c1_ref,
                      wl_ref, wr_ref, b_ref, g_ref, be_ref, batch_ref,
                      o_ref, gr_ref, sums, cnts, maxs):
    i = pl.program_id(0)

    @pl.when(i == 0)
    def _():
        sums[...] = jnp.zeros((G, D), jnp.float32)
        cnts[...] = jnp.zeros((G, D), jnp.float32)
        maxs[...] = jnp.full((G, D), -jnp.inf, jnp.float32)

    cnt = jnp.clip(c0_ref[...] + c1_ref[...], 1.0, None)
    nmean = (p0_ref[...] + p1_ref[...]) / cnt
    y = (jnp.dot(h_ref[...], wl_ref[...], preferred_element_type=jnp.float32)
         + jnp.dot(nmean, wr_ref[...], preferred_element_type=jnp.float32))
    y = _ln_relu(y + b_ref[...], g_ref[...], be_ref[...])
    o_ref[...] = y

    b_blk = batch_ref[0]
    for g in range(G):
        m = b_blk == g
        sums[g, :] += jnp.sum(jnp.where(m, y, 0.0), axis=0)
        cnts[g, :] += jnp.sum(jnp.where(m, 1.0, 0.0) * jnp.ones((1, D)), axis=0)
        maxs[g, :] = jnp.maximum(maxs[g, :],
                                 jnp.max(jnp.where(m, y, -jnp.inf), axis=0))

    @pl.when(i == _GRID - 1)
    def _():
        gr_ref[:, :D] = sums[...] / jnp.clip(cnts[...], 1.0, None)
        gr_ref[:, D:] = maxs[...]


_row_spec = pl.BlockSpec((_BLK, D), lambda i: (i, 0))
_full_spec = lambda r, c: pl.BlockSpec((r, c), lambda i: (0, 0))
_col_spec = pl.BlockSpec((_BLK, 1), lambda i: (i, 0))

_dense0 = pl.pallas_call(
    _dense0_kernel,
    grid=(_GRID,),
    in_specs=[_row_spec, _full_spec(D, D), _full_spec(1, D),
              _full_spec(1, D), _full_spec(1, D)],
    out_specs=_row_spec,
    out_shape=jax.ShapeDtypeStruct((N, D), jnp.float32),
)

_sage = pl.pallas_call(
    _sage_kernel,
    grid=(_GRID,),
    in_specs=[_row_spec, _row_spec, _row_spec, _col_spec, _col_spec,
              _full_spec(D, D), _full_spec(D, D), _full_spec(1, D),
              _full_spec(1, D), _full_spec(1, D)],
    out_specs=_row_spec,
    out_shape=jax.ShapeDtypeStruct((N, D), jnp.float32),
)

_sage_pool = pl.pallas_call(
    _sage_pool_kernel,
    grid=(_GRID,),
    in_specs=[_row_spec, _row_spec, _row_spec, _col_spec, _col_spec,
              _full_spec(D, D), _full_spec(D, D), _full_spec(1, D),
              _full_spec(1, D), _full_spec(1, D),
              pl.BlockSpec((1, _BLK, 1), lambda i: (i, 0, 0))],
    out_specs=[_row_spec, pl.BlockSpec((G, 2 * D), lambda i: (0, 0))],
    out_shape=[jax.ShapeDtypeStruct((N, D), jnp.float32),
               jax.ShapeDtypeStruct((G, 2 * D), jnp.float32)],
    scratch_shapes=[pltpu.VMEM((G, D), jnp.float32),
                    pltpu.VMEM((G, D), jnp.float32),
                    pltpu.VMEM((G, D), jnp.float32)],
)


def kernel(x, edge_index, batch, W0, b0, g0, be0, W1, b1, g1, be1,
           W2, b2, g2, be2, W3, b3, g3, be3):
    src = edge_index[0].astype(jnp.int32).reshape(NW, EDGES_PER_TILE)
    dst = edge_index[1].astype(jnp.int32).reshape(NW, EDGES_PER_TILE)
    src_p = jnp.pad(src, ((0, 0), (0, EDGES_PAD))).reshape(NW, NCHUNK, CHUNK)
    dst_p = jnp.pad(dst, ((0, 0), (0, EDGES_PAD)),
                    constant_values=TRASH).reshape(NW, NCHUNK, CHUNK)

    idx = jnp.stack([src_p, dst_p], axis=2)
    idx = idx.reshape(NW, 2 * NCHUNK, CHUNK)
    dst_c = edge_index[1].astype(jnp.int32).reshape(NW, NCHUNK_C, CHUNK_C)
    zeros2d = jnp.zeros((NACC, D), jnp.float32)
    zeros1d = jnp.zeros((N,), jnp.float32)
    batch3 = batch.astype(jnp.int32).reshape(_GRID, _BLK, 1)

    def row(v):
        return v.reshape(1, D)

    _agg, _cnt = _sc_kernels()
    cp = _cnt(dst_c, zeros1d)
    c0 = cp[0, 0].reshape(N, 1)
    c1 = cp[1, 0].reshape(N, 1)

    h = _dense0(x, W0.T, row(b0), row(g0), row(be0))

    def sage(h, W, b, g, be):
        p = _agg(h, idx, zeros2d)
        return _sage(h, p[0], p[1], c0, c1, W[:, :D].T, W[:, D:].T,
                     row(b), row(g), row(be))

    h = sage(h, W1, b1, g1, be1)
    h = sage(h, W2, b2, g2, be2)
    p = _agg(h, idx, zeros2d)
    node_embed, graph_embed = _sage_pool(
        h, p[0], p[1], c0, c1, W3[:, :D].T, W3[:, D:].T,
        row(b3), row(g3), row(be3), batch3)
    return (node_embed, graph_embed)


# speedup vs baseline: 1.1191x; 1.1191x over previous
import functools

import jax
import jax.numpy as jnp
from jax import lax
from jax.experimental import pallas as pl
from jax.experimental.pallas import tpu as pltpu
from jax.experimental.pallas import tpu_sc as plsc

N = 10000
E = 320000
D = 128
G = 8

NC = 2
NS = 16
NW = NC * NS
EDGES_PER_TILE = E // NW
CHUNK = 128
NCHUNK = -(-EDGES_PER_TILE // CHUNK)
EDGES_PAD = NCHUNK * CHUNK - EDGES_PER_TILE
NACC = N + NW
CHUNK_C = 80
NCHUNK_C = EDGES_PER_TILE // CHUNK_C
ROWS_PER_TILE = 624
ZERO_REM = NACC - NS * ROWS_PER_TILE
OUT_REM = N - NS * ROWS_PER_TILE

_BLK = 1000
_GRID = N // _BLK


def _agg_body(h_hbm, idx_hbm, zeros_hbm, out_hbm,
              ibuf, rows_0, rows_1, rows_2, accum,
              isem_0, isem_1, isem_2, gsem_0, gsem_1, gsem_2,
              ssem_0, ssem_1, ssem_2):
    rows = (rows_0, rows_1, rows_2)
    isem = (isem_0, isem_1, isem_2)
    gsem = (gsem_0, gsem_1, gsem_2)
    ssem = (ssem_0, ssem_1, ssem_2)
    cid = lax.axis_index("c")
    sid = lax.axis_index("s")
    wid = cid * NS + sid

    pltpu.sync_copy(zeros_hbm.at[pl.ds(sid * ROWS_PER_TILE, ROWS_PER_TILE)],
                    accum.at[pl.ds(sid * ROWS_PER_TILE, ROWS_PER_TILE)])

    @pl.when(sid == NS - 1)
    def _():
        pltpu.sync_copy(zeros_hbm.at[pl.ds(NS * ROWS_PER_TILE, ZERO_REM)],
                        accum.at[pl.ds(NS * ROWS_PER_TILE, ZERO_REM)])

    plsc.subcore_barrier()

    def stage_idx(c, b):
        dstsl = ibuf.at[pl.ds(2 * b, 2)]
        pltpu.async_copy(idx_hbm.at[wid, c], dstsl, isem[b])
        pltpu.make_async_copy(idx_hbm.at[wid, c], dstsl, isem[b]).wait()

    def start_gather(b):
        pltpu.async_copy(h_hbm.at[ibuf.at[2 * b]], rows[b], gsem[b])

    def wait_gather(b):
        pltpu.make_async_copy(h_hbm.at[ibuf.at[2 * b]], rows[b],
                              gsem[b]).wait()

    def start_scatter(b):
        pltpu.async_copy(rows[b], accum.at[ibuf.at[2 * b + 1]], ssem[b],
                         add=True)

    def wait_scatter(b):
        pltpu.make_async_copy(rows[b], accum.at[ibuf.at[2 * b + 1]],
                              ssem[b]).wait()

    for c in (0, 1):
        stage_idx(c, c)
        start_gather(c)

    def body(m, carry):
        c0 = 3 * m
        for l in range(3):
            c = c0 + l
            b = l
            b2 = (l + 2) % 3

            @pl.when(c < NCHUNK)
            def _():
                wait_gather(b)
                start_scatter(b)

            @pl.when(c + 2 < NCHUNK)
            def _():
                @pl.when(c >= 1)
                def _():
                    wait_scatter(b2)
                stage_idx(c + 2, b2)
                start_gather(b2)

        return carry

    lax.fori_loop(0, (NCHUNK + 2) // 3, body, 0)
    for s in (NCHUNK - 3, NCHUNK - 2, NCHUNK - 1):
        wait_scatter(s % 3)
    plsc.subcore_barrier()
    pltpu.sync_copy(accum.at[pl.ds(sid * ROWS_PER_TILE, ROWS_PER_TILE)],
                    out_hbm.at[cid, pl.ds(sid * ROWS_PER_TILE, ROWS_PER_TILE)])

    @pl.when(sid == NS - 1)
    def _():
        pltpu.sync_copy(accum.at[pl.ds(NS * ROWS_PER_TILE, OUT_REM)],
                        out_hbm.at[cid, pl.ds(NS * ROWS_PER_TILE, OUT_REM)])


def _cnt_body(dst_hbm, zeros_hbm, out_hbm, idx_d, ones, cnt, sem):
    cid = lax.axis_index("c")
    sid = lax.axis_index("s")
    wid = cid * NS + sid

    @pl.when(sid == 0)
    def _():
        pltpu.sync_copy(zeros_hbm, cnt)
    for i in range(CHUNK_C // 16):
        ones[pl.ds(i * 16, 16)] = jnp.full((16,), 1.0, jnp.float32)
    pltpu.sync_copy(dst_hbm.at[wid], idx_d)
    plsc.subcore_barrier()

    def body(j, carry):
        pltpu.sync_copy(ones, cnt.at[idx_d.at[j]], add=True)
        return carry

    lax.fori_loop(0, NCHUNK_C, body, 0)
    plsc.subcore_barrier()

    @pl.when(sid == 0)
    def _():
        pltpu.sync_copy(cnt, out_hbm.at[cid, 0])


@functools.cache
def _sc_kernels():
    mesh = plsc.VectorSubcoreMesh(core_axis_name="c", subcore_axis_name="s",
                                  num_cores=NC, num_subcores=NS)
    agg = pl.kernel(
        _agg_body,
        out_type=jax.ShapeDtypeStruct((NC, N, D), jnp.float32),
        mesh=mesh,
        scratch_types=(
            [pltpu.VMEM((6, CHUNK), jnp.int32)]
            + [pltpu.VMEM((CHUNK, D), jnp.float32)] * 3
            + [pltpu.VMEM_SHARED((NACC, D), jnp.float32)]
            + [pltpu.SemaphoreType.DMA] * 9
        ),
    )
    cnt = pl.kernel(
        _cnt_body,
        out_type=jax.ShapeDtypeStruct((NC, 1, N), jnp.float32),
        mesh=mesh,
        scratch_types=[
            pltpu.VMEM((NCHUNK_C, CHUNK_C), jnp.int32),
            pltpu.VMEM((CHUNK_C,), jnp.float32),
            pltpu.VMEM_SHARED((N,), jnp.float32),
            pltpu.SemaphoreType.DMA,
        ],
    )
    return agg, cnt


def _ln_relu(y, g, be):
    m = jnp.mean(y, axis=-1, keepdims=True)
    v = jnp.mean((y - m) ** 2, axis=-1, keepdims=True)
    return jnp.maximum((y - m) * lax.rsqrt(v + 1e-5) * g + be, 0.0)


def _dense0_kernel(x_ref, w_ref, b_ref, g_ref, be_ref, o_ref):
    y = jnp.dot(x_ref[...], w_ref[...], preferred_element_type=jnp.float32)
    o_ref[...] = _ln_relu(y + b_ref[...], g_ref[...], be_ref[...])


def _sage_kernel(h_ref, p0_ref, p1_ref, c0_ref, c1_ref,
                 wl_ref, wr_ref, b_ref, g_ref, be_ref, o_ref):
    cnt = jnp.clip(c0_ref[...] + c1_ref[...], 1.0, None)
    nmean = (p0_ref[...] + p1_ref[...]) / cnt
    y = (jnp.dot(h_ref[...], wl_ref[...], preferred_element_type=jnp.float32)
         + jnp.dot(nmean, wr_ref[...], preferred_element_type=jnp.float32))
    o_ref[...] = _ln_relu(y + b_ref[...], g_ref[...], be_ref[...])


def _sage_pool_kernel(h_ref, p0_ref, p1_ref, c0_ref, c1_ref,
                      wl_ref, wr_ref, b_ref, g_ref, be_ref, batch_ref,
                      o_ref, gr_ref, sums, cnts, maxs):
    i = pl.program_id(0)

    @pl.when(i == 0)
    def _():
        sums[...] = jnp.zeros((G, D), jnp.float32)
        cnts[...] = jnp.zeros((G, D), jnp.float32)
        maxs[...] = jnp.full((G, D), -jnp.inf, jnp.float32)

    cnt = jnp.clip(c0_ref[...] + c1_ref[...], 1.0, None)
    nmean = (p0_ref[...] + p1_ref[...]) / cnt
    y = (jnp.dot(h_ref[...], wl_ref[...], preferred_element_type=jnp.float32)
         + jnp.dot(nmean, wr_ref[...], preferred_element_type=jnp.float32))
    y = _ln_relu(y + b_ref[...], g_ref[...], be_ref[...])
    o_ref[...] = y

    b_blk = batch_ref[0]
    for g in range(G):
        m = b_blk == g
        sums[g, :] += jnp.sum(jnp.where(m, y, 0.0), axis=0)
        cnts[g, :] += jnp.sum(jnp.where(m, 1.0, 0.0) * jnp.ones((1, D)), axis=0)
        maxs[g, :] = jnp.maximum(maxs[g, :],
                                 jnp.max(jnp.where(m, y, -jnp.inf), axis=0))

    @pl.when(i == _GRID - 1)
    def _():
        gr_ref[:, :D] = sums[...] / jnp.clip(cnts[...], 1.0, None)
        gr_ref[:, D:] = maxs[...]


_row_spec = pl.BlockSpec((_BLK, D), lambda i: (i, 0))
_full_spec = lambda r, c: pl.BlockSpec((r, c), lambda i: (0, 0))
_col_spec = pl.BlockSpec((_BLK, 1), lambda i: (i, 0))

_dense0 = pl.pallas_call(
    _dense0_kernel,
    grid=(_GRID,),
    in_specs=[_row_spec, _full_spec(D, D), _full_spec(1, D),
              _full_spec(1, D), _full_spec(1, D)],
    out_specs=_row_spec,
    out_shape=jax.ShapeDtypeStruct((N, D), jnp.float32),
)

_sage = pl.pallas_call(
    _sage_kernel,
    grid=(_GRID,),
    in_specs=[_row_spec, _row_spec, _row_spec, _col_spec, _col_spec,
              _full_spec(D, D), _full_spec(D, D), _full_spec(1, D),
              _full_spec(1, D), _full_spec(1, D)],
    out_specs=_row_spec,
    out_shape=jax.ShapeDtypeStruct((N, D), jnp.float32),
)

_sage_pool = pl.pallas_call(
    _sage_pool_kernel,
    grid=(_GRID,),
    in_specs=[_row_spec, _row_spec, _row_spec, _col_spec, _col_spec,
              _full_spec(D, D), _full_spec(D, D), _full_spec(1, D),
              _full_spec(1, D), _full_spec(1, D),
              pl.BlockSpec((1, _BLK, 1), lambda i: (i, 0, 0))],
    out_specs=[_row_spec, pl.BlockSpec((G, 2 * D), lambda i: (0, 0))],
    out_shape=[jax.ShapeDtypeStruct((N, D), jnp.float32),
               jax.ShapeDtypeStruct((G, 2 * D), jnp.float32)],
    scratch_shapes=[pltpu.VMEM((G, D), jnp.float32),
                    pltpu.VMEM((G, D), jnp.float32),
                    pltpu.VMEM((G, D), jnp.float32)],
)


def kernel(x, edge_index, batch, W0, b0, g0, be0, W1, b1, g1, be1,
           W2, b2, g2, be2, W3, b3, g3, be3):
    src = edge_index[0].astype(jnp.int32).reshape(NW, EDGES_PER_TILE)
    dst = edge_index[1].astype(jnp.int32).reshape(NW, EDGES_PER_TILE)
    src_p = jnp.pad(src, ((0, 0), (0, EDGES_PAD))).reshape(NW, NCHUNK, CHUNK)
    trash = jnp.broadcast_to(N + jnp.arange(NW, dtype=jnp.int32)[:, None],
                             (NW, EDGES_PAD))
    dst_p = jnp.concatenate([dst, trash], axis=1).reshape(NW, NCHUNK, CHUNK)
    idx = jnp.stack([src_p, dst_p], axis=2)
    dst_c = edge_index[1].astype(jnp.int32).reshape(NW, NCHUNK_C, CHUNK_C)
    zeros2d = jnp.zeros((NACC, D), jnp.float32)
    zeros1d = jnp.zeros((N,), jnp.float32)
    batch3 = batch.astype(jnp.int32).reshape(_GRID, _BLK, 1)

    def row(v):
        return v.reshape(1, D)

    _agg, _cnt = _sc_kernels()
    cp = _cnt(dst_c, zeros1d)
    c0 = cp[0, 0].reshape(N, 1)
    c1 = cp[1, 0].reshape(N, 1)

    h = _dense0(x, W0.T, row(b0), row(g0), row(be0))

    def sage(h, W, b, g, be):
        p = _agg(h, idx, zeros2d)
        return _sage(h, p[0], p[1], c0, c1, W[:, :D].T, W[:, D:].T,
                     row(b), row(g), row(be))

    h = sage(h, W1, b1, g1, be1)
    h = sage(h, W2, b2, g2, be2)
    p = _agg(h, idx, zeros2d)
    node_embed, graph_embed = _sage_pool(
        h, p[0], p[1], c0, c1, W3[:, :D].T, W3[:, D:].T,
        row(b3), row(g3), row(be3), batch3)
    return (node_embed, graph_embed)


# speedup vs baseline: 1.9535x; 1.7456x over previous
import functools

import jax
import jax.numpy as jnp
from jax import lax
from jax.experimental import pallas as pl
from jax.experimental.pallas import tpu as pltpu
from jax.experimental.pallas import tpu_sc as plsc

N = 10000
E = 320000
D = 128
G = 8

NC = 2
NS = 16
NW = NC * NS
EDGES_PER_TILE = E // NW
CHUNK = 100
NCHUNK = -(-EDGES_PER_TILE // CHUNK)
EDGES_PAD = NCHUNK * CHUNK - EDGES_PER_TILE
NACC = N + NW
CHUNK_C = 80
NCHUNK_C = EDGES_PER_TILE // CHUNK_C
ROWS_PER_TILE = 624
ZERO_REM = NACC - NS * ROWS_PER_TILE
OUT_REM = N - NS * ROWS_PER_TILE

_BLK = 1000
_GRID = N // _BLK


def _agg_body(h_hbm, idx_hbm, zeros_hbm, out_hbm,
              ibuf, rows_0, rows_1, rows_2, accum,
              isem_0, isem_1, isem_2, gsem_0, gsem_1, gsem_2,
              ssem_0, ssem_1, ssem_2):
    rows = (rows_0, rows_1, rows_2)
    isem = (isem_0, isem_1, isem_2)
    gsem = (gsem_0, gsem_1, gsem_2)
    ssem = (ssem_0, ssem_1, ssem_2)
    cid = lax.axis_index("c")
    sid = lax.axis_index("s")
    wid = cid * NS + sid

    pltpu.sync_copy(zeros_hbm.at[pl.ds(sid * ROWS_PER_TILE, ROWS_PER_TILE)],
                    accum.at[pl.ds(sid * ROWS_PER_TILE, ROWS_PER_TILE)])

    @pl.when(sid == NS - 1)
    def _():
        pltpu.sync_copy(zeros_hbm.at[pl.ds(NS * ROWS_PER_TILE, ZERO_REM)],
                        accum.at[pl.ds(NS * ROWS_PER_TILE, ZERO_REM)])

    plsc.subcore_barrier()

    def stage_idx(c, b):
        dstsl = ibuf.at[pl.ds(2 * b, 2)]
        pltpu.async_copy(idx_hbm.at[wid, c], dstsl, isem[b])
        pltpu.make_async_copy(idx_hbm.at[wid, c], dstsl, isem[b]).wait()

    def start_gather(b):
        pltpu.async_copy(h_hbm.at[ibuf.at[2 * b]], rows[b], gsem[b])

    def wait_gather(b):
        pltpu.make_async_copy(h_hbm.at[ibuf.at[2 * b]], rows[b],
                              gsem[b]).wait()

    def start_scatter(b):
        pltpu.async_copy(rows[b], accum.at[ibuf.at[2 * b + 1]], ssem[b],
                         add=True)

    def wait_scatter(b):
        pltpu.make_async_copy(rows[b], accum.at[ibuf.at[2 * b + 1]],
                              ssem[b]).wait()

    for c in (0, 1):
        stage_idx(c, c)
        start_gather(c)

    def body(m, carry):
        c0 = 3 * m
        for l in range(3):
            c = c0 + l
            b = l
            b2 = (l + 2) % 3

            @pl.when(c < NCHUNK)
            def _():
                wait_gather(b)
                start_scatter(b)

            @pl.when(c + 2 < NCHUNK)
            def _():
                @pl.when(c >= 1)
                def _():
                    wait_scatter(b2)
                stage_idx(c + 2, b2)
                start_gather(b2)

        return carry

    lax.fori_loop(0, (NCHUNK + 2) // 3, body, 0)
    for s in (NCHUNK - 3, NCHUNK - 2, NCHUNK - 1):
        wait_scatter(s % 3)
    plsc.subcore_barrier()
    pltpu.sync_copy(accum.at[pl.ds(sid * ROWS_PER_TILE, ROWS_PER_TILE)],
                    out_hbm.at[cid, pl.ds(sid * ROWS_PER_TILE, ROWS_PER_TILE)])

    @pl.when(sid == NS - 1)
    def _():
        pltpu.sync_copy(accum.at[pl.ds(NS * ROWS_PER_TILE, OUT_REM)],
                        out_hbm.at[cid, pl.ds(NS * ROWS_PER_TILE, OUT_REM)])


def _cnt_body(dst_hbm, zeros_hbm, out_hbm, idx_d, ones, cnt, sem):
    cid = lax.axis_index("c")
    sid = lax.axis_index("s")
    wid = cid * NS + sid

    @pl.when(sid == 0)
    def _():
        pltpu.sync_copy(zeros_hbm, cnt)
    for i in range(CHUNK_C // 16):
        ones[pl.ds(i * 16, 16)] = jnp.full((16,), 1.0, jnp.float32)
    pltpu.sync_copy(dst_hbm.at[wid], idx_d)
    plsc.subcore_barrier()

    def body(j, carry):
        pltpu.sync_copy(ones, cnt.at[idx_d.at[j]], add=True)
        return carry

    lax.fori_loop(0, NCHUNK_C, body, 0)
    plsc.subcore_barrier()

    @pl.when(sid == 0)
    def _():
        pltpu.sync_copy(cnt, out_hbm.at[cid, 0])


@functools.cache
def _sc_kernels():
    mesh = plsc.VectorSubcoreMesh(core_axis_name="c", subcore_axis_name="s",
                                  num_cores=NC, num_subcores=NS)
    agg = pl.kernel(
        _agg_body,
        out_type=jax.ShapeDtypeStruct((NC, N, D), jnp.float32),
        mesh=mesh,
        scratch_types=(
            [pltpu.VMEM((6, CHUNK), jnp.int32)]
            + [pltpu.VMEM((CHUNK, D), jnp.float32)] * 3
            + [pltpu.VMEM_SHARED((NACC, D), jnp.float32)]
            + [pltpu.SemaphoreType.DMA] * 9
        ),
    )
    cnt = pl.kernel(
        _cnt_body,
        out_type=jax.ShapeDtypeStruct((NC, 1, N), jnp.float32),
        mesh=mesh,
        scratch_types=[
            pltpu.VMEM((NCHUNK_C, CHUNK_C), jnp.int32),
            pltpu.VMEM((CHUNK_C,), jnp.float32),
            pltpu.VMEM_SHARED((N,), jnp.float32),
            pltpu.SemaphoreType.DMA,
        ],
    )
    return agg, cnt


def _ln_relu(y, g, be):
    m = jnp.mean(y, axis=-1, keepdims=True)
    v = jnp.mean((y - m) ** 2, axis=-1, keepdims=True)
    return jnp.maximum((y - m) * lax.rsqrt(v + 1e-5) * g + be, 0.0)


def _dense0_kernel(x_ref, w_ref, b_ref, g_ref, be_ref, o_ref):
    y = jnp.dot(x_ref[...], w_ref[...], preferred_element_type=jnp.float32)
    o_ref[...] = _ln_relu(y + b_ref[...], g_ref[...], be_ref[...])


def _sage_kernel(h_ref, p0_ref, p1_ref, c0_ref, c1_ref,
                 wl_ref, wr_ref, b_ref, g_ref, be_ref, o_ref):
    cnt = jnp.clip(c0_ref[...] + c1_ref[...], 1.0, None)
    nmean = (p0_ref[...] + p1_ref[...]) / cnt
    y = (jnp.dot(h_ref[...], wl_ref[...], preferred_element_type=jnp.float32)
         + jnp.dot(nmean, wr_ref[...], preferred_element_type=jnp.float32))
    o_ref[...] = _ln_relu(y + b_ref[...], g_ref[...], be_ref[...])


def _sage_pool_kernel(h_ref, p0_ref, p1_ref, c0_ref, c1_ref,
                      wl_ref, wr_ref, b_ref, g_ref, be_ref, batch_ref,
                      o_ref, gr_ref, sums, cnts, maxs):
    i = pl.program_id(0)

    @pl.when(i == 0)
    def _():
        sums[...] = jnp.zeros((G, D), jnp.float32)
        cnts[...] = jnp.zeros((G, D), jnp.float32)
        maxs[...] = jnp.full((G, D), -jnp.inf, jnp.float32)

    cnt = jnp.clip(c0_ref[...] + c1_ref[...], 1.0, None)
    nmean = (p0_ref[...] + p1_ref[...]) / cnt
    y = (jnp.dot(h_ref[...], wl_ref[...], preferred_element_type=jnp.float32)
         + jnp.dot(nmean, wr_ref[...], preferred_element_type=jnp.float32))
    y = _ln_relu(y + b_ref[...], g_ref[...], be_ref[...])
    o_ref[...] = y

    b_blk = batch_ref[0]
    for g in range(G):
        m = b_blk == g
        sums[g, :] += jnp.sum(jnp.where(m, y, 0.0), axis=0)
        cnts[g, :] += jnp.sum(jnp.where(m, 1.0, 0.0) * jnp.ones((1, D)), axis=0)
        maxs[g, :] = jnp.maximum(maxs[g, :],
                                 jnp.max(jnp.where(m, y, -jnp.inf), axis=0))

    @pl.when(i == _GRID - 1)
    def _():
        gr_ref[:, :D] = sums[...] / jnp.clip(cnts[...], 1.0, None)
        gr_ref[:, D:] = maxs[...]


_row_spec = pl.BlockSpec((_BLK, D), lambda i: (i, 0))
_full_spec = lambda r, c: pl.BlockSpec((r, c), lambda i: (0, 0))
_col_spec = pl.BlockSpec((_BLK, 1), lambda i: (i, 0))

_dense0 = pl.pallas_call(
    _dense0_kernel,
    grid=(_GRID,),
    in_specs=[_row_spec, _full_spec(D, D), _full_spec(1, D),
              _full_spec(1, D), _full_spec(1, D)],
    out_specs=_row_spec,
    out_shape=jax.ShapeDtypeStruct((N, D), jnp.float32),
)

_sage = pl.pallas_call(
    _sage_kernel,
    grid=(_GRID,),
    in_specs=[_row_spec, _row_spec, _row_spec, _col_spec, _col_spec,
              _full_spec(D, D), _full_spec(D, D), _full_spec(1, D),
              _full_spec(1, D), _full_spec(1, D)],
    out_specs=_row_spec,
    out_shape=jax.ShapeDtypeStruct((N, D), jnp.float32),
)

_sage_pool = pl.pallas_call(
    _sage_pool_kernel,
    grid=(_GRID,),
    in_specs=[_row_spec, _row_spec, _row_spec, _col_spec, _col_spec,
              _full_spec(D, D), _full_spec(D, D), _full_spec(1, D),
              _full_spec(1, D), _full_spec(1, D),
              pl.BlockSpec((1, _BLK, 1), lambda i: (i, 0, 0))],
    out_specs=[_row_spec, pl.BlockSpec((G, 2 * D), lambda i: (0, 0))],
    out_shape=[jax.ShapeDtypeStruct((N, D), jnp.float32),
               jax.ShapeDtypeStruct((G, 2 * D), jnp.float32)],
    scratch_shapes=[pltpu.VMEM((G, D), jnp.float32),
                    pltpu.VMEM((G, D), jnp.float32),
                    pltpu.VMEM((G, D), jnp.float32)],
)


def kernel(x, edge_index, batch, W0, b0, g0, be0, W1, b1, g1, be1,
           W2, b2, g2, be2, W3, b3, g3, be3):
    src = edge_index[0].astype(jnp.int32).reshape(NW, EDGES_PER_TILE)
    dst = edge_index[1].astype(jnp.int32).reshape(NW, EDGES_PER_TILE)
    src_p = jnp.pad(src, ((0, 0), (0, EDGES_PAD))).reshape(NW, NCHUNK, CHUNK)
    trash = jnp.broadcast_to(N + jnp.arange(NW, dtype=jnp.int32)[:, None],
                             (NW, EDGES_PAD))
    dst_p = jnp.concatenate([dst, trash], axis=1).reshape(NW, NCHUNK, CHUNK)
    idx = jnp.stack([src_p, dst_p], axis=2)
    dst_c = edge_index[1].astype(jnp.int32).reshape(NW, NCHUNK_C, CHUNK_C)
    zeros2d = jnp.zeros((NACC, D), jnp.float32)
    zeros1d = jnp.zeros((N,), jnp.float32)
    batch3 = batch.astype(jnp.int32).reshape(_GRID, _BLK, 1)

    def row(v):
        return v.reshape(1, D)

    _agg, _cnt = _sc_kernels()
    cp = _cnt(dst_c, zeros1d)
    c0 = cp[0, 0].reshape(N, 1)
    c1 = cp[1, 0].reshape(N, 1)

    h = _dense0(x, W0.T, row(b0), row(g0), row(be0))

    def sage(h, W, b, g, be):
        p = _agg(h, idx, zeros2d)
        return _sage(h, p[0], p[1], c0, c1, W[:, :D].T, W[:, D:].T,
                     row(b), row(g), row(be))

    h = sage(h, W1, b1, g1, be1)
    h = sage(h, W2, b2, g2, be2)
    p = _agg(h, idx, zeros2d)
    node_embed, graph_embed = _sage_pool(
        h, p[0], p[1], c0, c1, W3[:, :D].T, W3[:, D:].T,
        row(b3), row(g3), row(be3), batch3)
    return (node_embed, graph_embed)
